# Initial kernel scaffold; baseline (speedup 1.0000x reference)
import jax
import jax.numpy as jnp
from jax.experimental import pallas as pl


def kernel(hyperedge_index, coef, rhs, W_rhs, b_rhs, W_c, b_c):
    raise NotImplementedError("write your pallas kernel here")


# speedup vs baseline: 89.6310x; 89.6310x over previous
import functools

import jax
import jax.numpy as jnp
from jax import lax
from jax.experimental import pallas as pl
from jax.experimental.pallas import tpu as pltpu
from jax.experimental.pallas import tpu_sc as plsc

NC = 2
NS = 16
L = 16
NW = NC * NS

NPAD = 10240

_SC_PARAMS = pltpu.CompilerParams(needs_layout_passes=False)
_SC_MESH = plsc.VectorSubcoreMesh(core_axis_name="c", subcore_axis_name="s")


def _worker(cid, sid):
    return sid * NC + cid


def _zero_acc(acc_ref, n):
    def body(i, _):
        acc_ref[pl.ds(i * L, L)] = jnp.zeros((L,), jnp.float32)
        return 0

    lax.fori_loop(0, n // L, body, 0, unroll=4)


def _pass_a_body(e_per, coef_hbm, col_hbm, sabs_out, csum_out,
                 coef_v, col_v, acc_s, acc_c):
    wid = _worker(lax.axis_index("c"), lax.axis_index("s"))
    base = wid * e_per
    pltpu.sync_copy(coef_hbm.at[pl.ds(base, e_per)], coef_v)
    pltpu.sync_copy(col_hbm.at[pl.ds(base, e_per)], col_v)
    _zero_acc(acc_s, NPAD)
    _zero_acc(acc_c, NPAD)

    def body(i, _):
        c = coef_v[pl.ds(i * L, L)]
        idx = col_v[pl.ds(i * L, L)]
        plsc.addupdate_scatter(acc_s, [idx], jnp.abs(c))
        plsc.addupdate_scatter(acc_c, [idx], c)
        return 0

    lax.fori_loop(0, e_per // L, body, 0, unroll=4)
    pltpu.sync_copy(acc_s, sabs_out.at[wid])
    pltpu.sync_copy(acc_c, csum_out.at[wid])


def _pass_b_body(e_per, coef_hbm, row_hbm, col_hbm, inv_hbm, rhss_hbm,
                 sv_out, u_out, coef_v, row_v, col_v, inv_v, rhss_v,
                 acc_sv, acc_u):
    wid = _worker(lax.axis_index("c"), lax.axis_index("s"))
    base = wid * e_per
    pltpu.sync_copy(coef_hbm.at[pl.ds(base, e_per)], coef_v)
    pltpu.sync_copy(row_hbm.at[pl.ds(base, e_per)], row_v)
    pltpu.sync_copy(col_hbm.at[pl.ds(base, e_per)], col_v)
    pltpu.sync_copy(inv_hbm, inv_v)
    pltpu.sync_copy(rhss_hbm, rhss_v)
    _zero_acc(acc_sv, NPAD)
    _zero_acc(acc_u, NPAD)

    def body(i, _):
        c = coef_v[pl.ds(i * L, L)]
        r = row_v[pl.ds(i * L, L)]
        cl = col_v[pl.ds(i * L, L)]
        scaled = c * plsc.load_gather(inv_v, [cl])
        plsc.addupdate_scatter(acc_sv, [r], scaled)
        plsc.addupdate_scatter(acc_u, [r], scaled * plsc.load_gather(rhss_v, [cl]))
        return 0

    lax.fori_loop(0, e_per // L, body, 0, unroll=4)
    pltpu.sync_copy(acc_sv, sv_out.at[wid])
    pltpu.sync_copy(acc_u, u_out.at[wid])


def _pass_c_body(e_per, coef_hbm, row_hbm, col_hbm, xv0_hbm, v_out,
                 coef_v, row_v, col_v, xv0_v, acc_v):
    wid = _worker(lax.axis_index("c"), lax.axis_index("s"))
    base = wid * e_per
    pltpu.sync_copy(coef_hbm.at[pl.ds(base, e_per)], coef_v)
    pltpu.sync_copy(row_hbm.at[pl.ds(base, e_per)], row_v)
    pltpu.sync_copy(col_hbm.at[pl.ds(base, e_per)], col_v)
    pltpu.sync_copy(xv0_hbm, xv0_v)
    _zero_acc(acc_v, NPAD)

    def body(i, _):
        c = coef_v[pl.ds(i * L, L)]
        r = row_v[pl.ds(i * L, L)]
        cl = col_v[pl.ds(i * L, L)]
        plsc.addupdate_scatter(acc_v, [cl], c * plsc.load_gather(xv0_v, [r]))
        return 0

    lax.fori_loop(0, e_per // L, body, 0, unroll=4)
    pltpu.sync_copy(acc_v, v_out.at[wid])


def _make_sc_kernels(e_per):
    f32 = jnp.float32
    i32 = jnp.int32
    pass_a = pl.kernel(
        functools.partial(_pass_a_body, e_per),
        out_type=[jax.ShapeDtypeStruct((NW, NPAD), f32)] * 2,
        mesh=_SC_MESH,
        compiler_params=_SC_PARAMS,
        scratch_types=[
            pltpu.VMEM((e_per,), f32),
            pltpu.VMEM((e_per,), i32),
            pltpu.VMEM((NPAD,), f32),
            pltpu.VMEM((NPAD,), f32),
        ],
        name="gnn_milp_pass_a",
    )
    pass_b = pl.kernel(
        functools.partial(_pass_b_body, e_per),
        out_type=[jax.ShapeDtypeStruct((NW, NPAD), f32)] * 2,
        mesh=_SC_MESH,
        compiler_params=_SC_PARAMS,
        scratch_types=[
            pltpu.VMEM((e_per,), f32),
            pltpu.VMEM((e_per,), i32),
            pltpu.VMEM((e_per,), i32),
            pltpu.VMEM((NPAD,), f32),
            pltpu.VMEM((NPAD,), f32),
            pltpu.VMEM((NPAD,), f32),
            pltpu.VMEM((NPAD,), f32),
        ],
        name="gnn_milp_pass_b",
    )
    pass_c = pl.kernel(
        functools.partial(_pass_c_body, e_per),
        out_type=[jax.ShapeDtypeStruct((NW, NPAD), f32)],
        mesh=_SC_MESH,
        compiler_params=_SC_PARAMS,
        scratch_types=[
            pltpu.VMEM((e_per,), f32),
            pltpu.VMEM((e_per,), i32),
            pltpu.VMEM((e_per,), i32),
            pltpu.VMEM((NPAD,), f32),
            pltpu.VMEM((NPAD,), f32),
        ],
        name="gnn_milp_pass_c",
    )
    return pass_a, pass_b, pass_c


def _safe_recip(s):
    inv = 1.0 / s
    return jnp.where(jnp.isinf(inv), 0.0, inv)


def _g1_body(sabs_ref, csum_ref, rhs_ref, inv_ref, t2_ref, rhss_ref):
    s = jnp.sum(sabs_ref[...], axis=0, keepdims=True)
    cs = jnp.sum(csum_ref[...], axis=0, keepdims=True)
    inv = _safe_recip(s)
    inv_ref[...] = inv
    t2_ref[...] = inv * cs
    rhss_ref[...] = inv * rhs_ref[...]


def _g2_body(sv_ref, u_ref, invv_ref, xv0_ref):
    s = jnp.sum(sv_ref[...], axis=0, keepdims=True)
    u = jnp.sum(u_ref[...], axis=0, keepdims=True)
    inv = _safe_recip(s)
    invv_ref[...] = inv
    xv0_ref[...] = inv * u


def _g3_body(n, v_ref, inv_ref, t2_ref, rhss_ref, wc_ref, bc_ref, w_ref, b_ref,
             t1_ref, rv_ref):
    v = jnp.sum(v_ref[...], axis=0, keepdims=True)
    t1 = inv_ref[...] * v
    t1_ref[...] = t1
    m1 = jnp.sum(t1) / n
    m2 = jnp.sum(t2_ref[...]) / n
    srhs = jnp.sum(rhss_ref[...])
    w = w_ref[...]
    b = b_ref[...]
    mean_vec = m1 * w + m2 * b
    aggr = lax.dot_general(mean_vec, wc_ref[...],
                           (((1,), (1,)), ((), ()))) + bc_ref[...]
    rv_ref[...] = (srhs * w + b) - aggr


def _expand_body(rhss_ref, xv0_ref, invv_ref, t1_ref, t2_ref, w_ref, b_ref,
                 rv_ref, emb_ref, xvar_ref, xconst_ref):
    w = w_ref[...]
    b = b_ref[...]
    rv = rv_ref[...]
    emb_ref[...] = rhss_ref[...] * w + b
    xvar_ref[...] = rv * (invv_ref[...] * w + b) + xv0_ref[...] * w + b
    xconst_ref[...] = t1_ref[...] * w + t2_ref[...] * b


def kernel(hyperedge_index, coef, rhs, W_rhs, b_rhs, W_c, b_c):
    f32 = jnp.float32
    nnz = coef.shape[0]
    n = rhs.shape[0]
    D = W_rhs.shape[0]
    e_per = nnz // NW
    assert nnz % NW == 0 and n <= NPAD

    row = hyperedge_index[0].astype(jnp.int32)
    col = hyperedge_index[1].astype(jnp.int32)
    coef = coef.astype(f32)

    pass_a, pass_b, pass_c = _make_sc_kernels(e_per)

    sabs_p, csum_p = pass_a(coef, col)
    rhs_pad = jnp.pad(rhs[:, 0].astype(f32), (0, NPAD - n)).reshape(1, NPAD)
    inv_sc, t2, rhs_s = pl.pallas_call(
        _g1_body,
        out_shape=[jax.ShapeDtypeStruct((1, NPAD), f32)] * 3,
    )(sabs_p, csum_p, rhs_pad)

    sv_p, u_p = pass_b(coef, row, col, inv_sc.reshape(NPAD), rhs_s.reshape(NPAD))
    inv_sv, xv0 = pl.pallas_call(
        _g2_body,
        out_shape=[jax.ShapeDtypeStruct((1, NPAD), f32)] * 2,
    )(sv_p, u_p)

    (v_p,) = pass_c(coef, row, col, xv0.reshape(NPAD))
    w = W_rhs[:, 0].astype(f32).reshape(1, D)
    b = b_rhs.astype(f32).reshape(1, D)
    t1, rhs_vec = pl.pallas_call(
        functools.partial(_g3_body, float(n)),
        out_shape=[jax.ShapeDtypeStruct((1, NPAD), f32),
                   jax.ShapeDtypeStruct((1, D), f32)],
    )(v_p, inv_sc, t2, rhs_s, W_c.astype(f32), b_c.astype(f32).reshape(1, D),
      w, b)

    BR = 1024
    col_spec = pl.BlockSpec((BR, 1), lambda i: (i, 0))
    vec_spec = pl.BlockSpec((1, D), lambda i: (0, 0))
    out_spec = pl.BlockSpec((BR, D), lambda i: (i, 0))
    emb, xvar, xconst = pl.pallas_call(
        _expand_body,
        grid=(NPAD // BR,),
        in_specs=[col_spec] * 5 + [vec_spec] * 3,
        out_specs=[out_spec] * 3,
        out_shape=[jax.ShapeDtypeStruct((NPAD, D), f32)] * 3,
    )(rhs_s.reshape(NPAD, 1), xv0.reshape(NPAD, 1), inv_sv.reshape(NPAD, 1),
      t1.reshape(NPAD, 1), t2.reshape(NPAD, 1), w, b, rhs_vec)

    return (xvar[:n], xconst[:n], emb[:n])


# speedup vs baseline: 103.9728x; 1.1600x over previous
import functools

import jax
import jax.numpy as jnp
from jax import lax
from jax.experimental import pallas as pl
from jax.experimental.pallas import tpu as pltpu
from jax.experimental.pallas import tpu_sc as plsc

NC = 2
NS = 16
L = 16
NW = NC * NS

NPAD = 10240

_SC_PARAMS = pltpu.CompilerParams(needs_layout_passes=False)
_SC_MESH = plsc.VectorSubcoreMesh(core_axis_name="c", subcore_axis_name="s")


def _worker(cid, sid):
    return sid * NC + cid


def _zero_acc(acc_ref, n):
    def body(i, _):
        acc_ref[pl.ds(i * L, L)] = jnp.zeros((L,), jnp.float32)
        return 0

    lax.fori_loop(0, n // L, body, 0, unroll=4)


def _pass_a_body(e_per, coef_hbm, col_hbm, sabs_out, csum_out,
                 coef_v, col_v, acc_s, acc_c, sem):
    wid = _worker(lax.axis_index("c"), lax.axis_index("s"))
    base = wid * e_per
    d1 = pltpu.async_copy(coef_hbm.at[pl.ds(base, e_per)], coef_v, sem)
    d2 = pltpu.async_copy(col_hbm.at[pl.ds(base, e_per)], col_v, sem)
    _zero_acc(acc_s, NPAD)
    _zero_acc(acc_c, NPAD)
    d1.wait()
    d2.wait()

    def body(i, _):
        c = coef_v[pl.ds(i * L, L)]
        idx = col_v[pl.ds(i * L, L)]
        plsc.addupdate_scatter(acc_s, [idx], jnp.abs(c))
        plsc.addupdate_scatter(acc_c, [idx], c)
        return 0

    lax.fori_loop(0, e_per // L, body, 0, unroll=4)
    pltpu.sync_copy(acc_s, sabs_out.at[wid])
    pltpu.sync_copy(acc_c, csum_out.at[wid])


def _pass_b_body(e_per, coef_hbm, row_hbm, col_hbm, inv_hbm, rhss_hbm,
                 sv_out, u_out, coef_v, row_v, col_v, inv_v, rhss_v,
                 acc_sv, acc_u, sem):
    wid = _worker(lax.axis_index("c"), lax.axis_index("s"))
    base = wid * e_per
    ds = [pltpu.async_copy(coef_hbm.at[pl.ds(base, e_per)], coef_v, sem),
          pltpu.async_copy(row_hbm.at[pl.ds(base, e_per)], row_v, sem),
          pltpu.async_copy(col_hbm.at[pl.ds(base, e_per)], col_v, sem),
          pltpu.async_copy(inv_hbm, inv_v, sem),
          pltpu.async_copy(rhss_hbm, rhss_v, sem)]
    _zero_acc(acc_sv, NPAD)
    _zero_acc(acc_u, NPAD)
    for d in ds:
        d.wait()

    def body(i, _):
        c = coef_v[pl.ds(i * L, L)]
        r = row_v[pl.ds(i * L, L)]
        cl = col_v[pl.ds(i * L, L)]
        scaled = c * plsc.load_gather(inv_v, [cl])
        plsc.addupdate_scatter(acc_sv, [r], scaled)
        plsc.addupdate_scatter(acc_u, [r], scaled * plsc.load_gather(rhss_v, [cl]))
        return 0

    lax.fori_loop(0, e_per // L, body, 0, unroll=4)
    pltpu.sync_copy(acc_sv, sv_out.at[wid])
    pltpu.sync_copy(acc_u, u_out.at[wid])


def _pass_c_body(e_per, coef_hbm, row_hbm, col_hbm, xv0_hbm, v_out,
                 coef_v, row_v, col_v, xv0_v, acc_v, sem):
    wid = _worker(lax.axis_index("c"), lax.axis_index("s"))
    base = wid * e_per
    ds = [pltpu.async_copy(coef_hbm.at[pl.ds(base, e_per)], coef_v, sem),
          pltpu.async_copy(row_hbm.at[pl.ds(base, e_per)], row_v, sem),
          pltpu.async_copy(col_hbm.at[pl.ds(base, e_per)], col_v, sem),
          pltpu.async_copy(xv0_hbm, xv0_v, sem)]
    _zero_acc(acc_v, NPAD)
    for d in ds:
        d.wait()

    def body(i, _):
        c = coef_v[pl.ds(i * L, L)]
        r = row_v[pl.ds(i * L, L)]
        cl = col_v[pl.ds(i * L, L)]
        plsc.addupdate_scatter(acc_v, [cl], c * plsc.load_gather(xv0_v, [r]))
        return 0

    lax.fori_loop(0, e_per // L, body, 0, unroll=4)
    pltpu.sync_copy(acc_v, v_out.at[wid])


def _make_sc_kernels(e_per):
    f32 = jnp.float32
    i32 = jnp.int32
    pass_a = pl.kernel(
        functools.partial(_pass_a_body, e_per),
        out_type=[jax.ShapeDtypeStruct((NW, NPAD), f32)] * 2,
        mesh=_SC_MESH,
        compiler_params=_SC_PARAMS,
        scratch_types=[
            pltpu.VMEM((e_per,), f32),
            pltpu.VMEM((e_per,), i32),
            pltpu.VMEM((NPAD,), f32),
            pltpu.VMEM((NPAD,), f32),
            pltpu.SemaphoreType.DMA,
        ],
        name="gnn_milp_pass_a",
    )
    pass_b = pl.kernel(
        functools.partial(_pass_b_body, e_per),
        out_type=[jax.ShapeDtypeStruct((NW, NPAD), f32)] * 2,
        mesh=_SC_MESH,
        compiler_params=_SC_PARAMS,
        scratch_types=[
            pltpu.VMEM((e_per,), f32),
            pltpu.VMEM((e_per,), i32),
            pltpu.VMEM((e_per,), i32),
            pltpu.VMEM((NPAD,), f32),
            pltpu.VMEM((NPAD,), f32),
            pltpu.VMEM((NPAD,), f32),
            pltpu.VMEM((NPAD,), f32),
            pltpu.SemaphoreType.DMA,
        ],
        name="gnn_milp_pass_b",
    )
    pass_c = pl.kernel(
        functools.partial(_pass_c_body, e_per),
        out_type=[jax.ShapeDtypeStruct((NW, NPAD), f32)],
        mesh=_SC_MESH,
        compiler_params=_SC_PARAMS,
        scratch_types=[
            pltpu.VMEM((e_per,), f32),
            pltpu.VMEM((e_per,), i32),
            pltpu.VMEM((e_per,), i32),
            pltpu.VMEM((NPAD,), f32),
            pltpu.VMEM((NPAD,), f32),
            pltpu.SemaphoreType.DMA,
        ],
        name="gnn_milp_pass_c",
    )
    return pass_a, pass_b, pass_c


def _safe_recip(s):
    inv = 1.0 / s
    return jnp.where(jnp.isinf(inv), 0.0, inv)


def _g1_body(sabs_ref, csum_ref, rhs_ref, inv_ref, t2_ref, rhss_ref):
    s = jnp.sum(sabs_ref[...], axis=0, keepdims=True)
    cs = jnp.sum(csum_ref[...], axis=0, keepdims=True)
    inv = _safe_recip(s)
    inv_ref[...] = inv
    t2_ref[...] = inv * cs
    rhss_ref[...] = inv * rhs_ref[...]


def _g2_body(sv_ref, u_ref, invv_ref, xv0_ref):
    s = jnp.sum(sv_ref[...], axis=0, keepdims=True)
    u = jnp.sum(u_ref[...], axis=0, keepdims=True)
    inv = _safe_recip(s)
    invv_ref[...] = inv
    xv0_ref[...] = inv * u


def _g3_body(n, v_ref, inv_ref, t2_ref, rhss_ref, wc_ref, bc_ref, w_ref, b_ref,
             t1_ref, rv_ref):
    v = jnp.sum(v_ref[...], axis=0, keepdims=True)
    t1 = inv_ref[...] * v
    t1_ref[...] = t1
    m1 = jnp.sum(t1) / n
    m2 = jnp.sum(t2_ref[...]) / n
    srhs = jnp.sum(rhss_ref[...])
    w = w_ref[...]
    b = b_ref[...]
    mean_vec = m1 * w + m2 * b
    aggr = lax.dot_general(mean_vec, wc_ref[...],
                           (((1,), (1,)), ((), ()))) + bc_ref[...]
    rv_ref[...] = (srhs * w + b) - aggr


def _expand_body(rhss_ref, xv0_ref, invv_ref, t1_ref, t2_ref, w_ref, b_ref,
                 rv_ref, emb_ref, xvar_ref, xconst_ref):
    w = w_ref[...]
    b = b_ref[...]
    rv = rv_ref[...]
    emb_ref[...] = rhss_ref[...] * w + b
    xvar_ref[...] = rv * (invv_ref[...] * w + b) + xv0_ref[...] * w + b
    xconst_ref[...] = t1_ref[...] * w + t2_ref[...] * b


def kernel(hyperedge_index, coef, rhs, W_rhs, b_rhs, W_c, b_c):
    f32 = jnp.float32
    nnz = coef.shape[0]
    n = rhs.shape[0]
    D = W_rhs.shape[0]
    e_per = nnz // NW
    assert nnz % NW == 0 and n <= NPAD

    row = hyperedge_index[0].astype(jnp.int32)
    col = hyperedge_index[1].astype(jnp.int32)
    coef = coef.astype(f32)

    pass_a, pass_b, pass_c = _make_sc_kernels(e_per)

    sabs_p, csum_p = pass_a(coef, col)
    rhs_pad = jnp.pad(rhs[:, 0].astype(f32), (0, NPAD - n)).reshape(1, NPAD)
    inv_sc, t2, rhs_s = pl.pallas_call(
        _g1_body,
        out_shape=[jax.ShapeDtypeStruct((1, NPAD), f32)] * 3,
    )(sabs_p, csum_p, rhs_pad)

    sv_p, u_p = pass_b(coef, row, col, inv_sc.reshape(NPAD), rhs_s.reshape(NPAD))
    inv_sv, xv0 = pl.pallas_call(
        _g2_body,
        out_shape=[jax.ShapeDtypeStruct((1, NPAD), f32)] * 2,
    )(sv_p, u_p)

    (v_p,) = pass_c(coef, row, col, xv0.reshape(NPAD))
    w = W_rhs[:, 0].astype(f32).reshape(1, D)
    b = b_rhs.astype(f32).reshape(1, D)
    t1, rhs_vec = pl.pallas_call(
        functools.partial(_g3_body, float(n)),
        out_shape=[jax.ShapeDtypeStruct((1, NPAD), f32),
                   jax.ShapeDtypeStruct((1, D), f32)],
    )(v_p, inv_sc, t2, rhs_s, W_c.astype(f32), b_c.astype(f32).reshape(1, D),
      w, b)

    BR = 1000
    col_spec = pl.BlockSpec((BR, 1), lambda i: (i, 0))
    vec_spec = pl.BlockSpec((1, D), lambda i: (0, 0))
    out_spec = pl.BlockSpec((BR, D), lambda i: (i, 0))
    emb, xvar, xconst = pl.pallas_call(
        _expand_body,
        grid=(n // BR,),
        in_specs=[col_spec] * 5 + [vec_spec] * 3,
        out_specs=[out_spec] * 3,
        out_shape=[jax.ShapeDtypeStruct((n, D), f32)] * 3,
    )(rhs_s.reshape(NPAD, 1), xv0.reshape(NPAD, 1), inv_sv.reshape(NPAD, 1),
      t1.reshape(NPAD, 1), t2.reshape(NPAD, 1), w, b, rhs_vec)

    return (xvar, xconst, emb)


# speedup vs baseline: 137.8455x; 1.3258x over previous
import functools

import jax
import jax.numpy as jnp
from jax import lax
from jax.experimental import pallas as pl
from jax.experimental.pallas import tpu as pltpu
from jax.experimental.pallas import tpu_sc as plsc

NC = 2
NS = 16
L = 16
NW = NC * NS

NPAD = 10240

_SC_PARAMS = pltpu.CompilerParams(needs_layout_passes=False)
_SC_MESH = plsc.VectorSubcoreMesh(core_axis_name="c", subcore_axis_name="s")


def _worker(cid, sid):
    return sid * NC + cid


def _zero_acc(acc_ref, n):
    def body(i, _):
        acc_ref[pl.ds(i * L, L)] = jnp.zeros((L,), jnp.float32)
        return 0

    lax.fori_loop(0, n // L, body, 0, unroll=4)


def _pass_a_body(e_per, nnz, coef_hbm, he_hbm, sabs_out, csum_out,
                 coef_v, col_v, acc_s, acc_c, sem):
    wid = _worker(lax.axis_index("c"), lax.axis_index("s"))
    base = wid * e_per
    d1 = pltpu.async_copy(coef_hbm.at[pl.ds(base, e_per)], coef_v, sem)
    d2 = pltpu.async_copy(he_hbm.at[pl.ds(nnz + base, e_per)], col_v, sem)
    _zero_acc(acc_s, NPAD)
    _zero_acc(acc_c, NPAD)
    d1.wait()
    d2.wait()

    def body(i, _):
        c = coef_v[pl.ds(i * L, L)]
        idx = col_v[pl.ds(i * L, L)]
        plsc.addupdate_scatter(acc_s, [idx], jnp.abs(c))
        plsc.addupdate_scatter(acc_c, [idx], c)
        return 0

    lax.fori_loop(0, e_per // L, body, 0, unroll=4)
    pltpu.sync_copy(acc_s, sabs_out.at[wid])
    pltpu.sync_copy(acc_c, csum_out.at[wid])


def _pass_b_body(e_per, nnz, coef_hbm, he_hbm, inv_hbm, rhss_hbm,
                 sv_out, u_out, coef_v, row_v, col_v, inv_v, rhss_v,
                 acc_sv, acc_u, sem):
    wid = _worker(lax.axis_index("c"), lax.axis_index("s"))
    base = wid * e_per
    ds = [pltpu.async_copy(coef_hbm.at[pl.ds(base, e_per)], coef_v, sem),
          pltpu.async_copy(he_hbm.at[pl.ds(base, e_per)], row_v, sem),
          pltpu.async_copy(he_hbm.at[pl.ds(nnz + base, e_per)], col_v, sem),
          pltpu.async_copy(inv_hbm, inv_v, sem),
          pltpu.async_copy(rhss_hbm, rhss_v, sem)]
    _zero_acc(acc_sv, NPAD)
    _zero_acc(acc_u, NPAD)
    for d in ds:
        d.wait()

    def body(i, _):
        c = coef_v[pl.ds(i * L, L)]
        r = row_v[pl.ds(i * L, L)]
        cl = col_v[pl.ds(i * L, L)]
        scaled = c * plsc.load_gather(inv_v, [cl])
        plsc.addupdate_scatter(acc_sv, [r], scaled)
        plsc.addupdate_scatter(acc_u, [r], scaled * plsc.load_gather(rhss_v, [cl]))
        return 0

    lax.fori_loop(0, e_per // L, body, 0, unroll=4)
    pltpu.sync_copy(acc_sv, sv_out.at[wid])
    pltpu.sync_copy(acc_u, u_out.at[wid])


def _pass_c_body(e_per, nnz, coef_hbm, he_hbm, xv0_hbm, v_out,
                 coef_v, row_v, col_v, xv0_v, acc_v, sem):
    wid = _worker(lax.axis_index("c"), lax.axis_index("s"))
    base = wid * e_per
    ds = [pltpu.async_copy(coef_hbm.at[pl.ds(base, e_per)], coef_v, sem),
          pltpu.async_copy(he_hbm.at[pl.ds(base, e_per)], row_v, sem),
          pltpu.async_copy(he_hbm.at[pl.ds(nnz + base, e_per)], col_v, sem),
          pltpu.async_copy(xv0_hbm, xv0_v, sem)]
    _zero_acc(acc_v, NPAD)
    for d in ds:
        d.wait()

    def body(i, _):
        c = coef_v[pl.ds(i * L, L)]
        r = row_v[pl.ds(i * L, L)]
        cl = col_v[pl.ds(i * L, L)]
        plsc.addupdate_scatter(acc_v, [cl], c * plsc.load_gather(xv0_v, [r]))
        return 0

    lax.fori_loop(0, e_per // L, body, 0, unroll=4)
    pltpu.sync_copy(acc_v, v_out.at[wid])


def _make_sc_kernels(e_per, nnz):
    f32 = jnp.float32
    i32 = jnp.int32
    pass_a = pl.kernel(
        functools.partial(_pass_a_body, e_per, nnz),
        out_type=[jax.ShapeDtypeStruct((NW, NPAD), f32)] * 2,
        mesh=_SC_MESH,
        compiler_params=_SC_PARAMS,
        scratch_types=[
            pltpu.VMEM((e_per,), f32),
            pltpu.VMEM((e_per,), i32),
            pltpu.VMEM((NPAD,), f32),
            pltpu.VMEM((NPAD,), f32),
            pltpu.SemaphoreType.DMA,
        ],
        name="gnn_milp_pass_a",
    )
    pass_b = pl.kernel(
        functools.partial(_pass_b_body, e_per, nnz),
        out_type=[jax.ShapeDtypeStruct((NW, NPAD), f32)] * 2,
        mesh=_SC_MESH,
        compiler_params=_SC_PARAMS,
        scratch_types=[
            pltpu.VMEM((e_per,), f32),
            pltpu.VMEM((e_per,), i32),
            pltpu.VMEM((e_per,), i32),
            pltpu.VMEM((NPAD,), f32),
            pltpu.VMEM((NPAD,), f32),
            pltpu.VMEM((NPAD,), f32),
            pltpu.VMEM((NPAD,), f32),
            pltpu.SemaphoreType.DMA,
        ],
        name="gnn_milp_pass_b",
    )
    pass_c = pl.kernel(
        functools.partial(_pass_c_body, e_per, nnz),
        out_type=[jax.ShapeDtypeStruct((NW, NPAD), f32)],
        mesh=_SC_MESH,
        compiler_params=_SC_PARAMS,
        scratch_types=[
            pltpu.VMEM((e_per,), f32),
            pltpu.VMEM((e_per,), i32),
            pltpu.VMEM((e_per,), i32),
            pltpu.VMEM((NPAD,), f32),
            pltpu.VMEM((NPAD,), f32),
            pltpu.SemaphoreType.DMA,
        ],
        name="gnn_milp_pass_c",
    )
    return pass_a, pass_b, pass_c


def _safe_recip(s):
    inv = 1.0 / s
    return jnp.where(jnp.isinf(inv), 0.0, inv)


def _g1_body(sabs_ref, csum_ref, rhs_ref, inv_ref, t2_ref, rhss_ref):
    s = jnp.sum(sabs_ref[...], axis=0, keepdims=True)
    cs = jnp.sum(csum_ref[...], axis=0, keepdims=True)
    inv = _safe_recip(s)
    inv_ref[...] = inv
    t2_ref[...] = inv * cs
    rhss_ref[...] = inv * rhs_ref[...]


def _g2_body(sv_ref, u_ref, invv_ref, xv0_ref):
    s = jnp.sum(sv_ref[...], axis=0, keepdims=True)
    u = jnp.sum(u_ref[...], axis=0, keepdims=True)
    inv = _safe_recip(s)
    invv_ref[...] = inv
    xv0_ref[...] = inv * u


def _g3_body(n, v_ref, inv_ref, t2_ref, rhss_ref, wc_ref, bc_ref, w_ref, b_ref,
             t1_ref, rv_ref):
    v = jnp.sum(v_ref[...], axis=0, keepdims=True)
    t1 = inv_ref[...] * v
    t1_ref[...] = t1
    m1 = jnp.sum(t1) / n
    m2 = jnp.sum(t2_ref[...]) / n
    srhs = jnp.sum(rhss_ref[...])
    w = w_ref[...]
    b = b_ref[...]
    mean_vec = m1 * w + m2 * b
    aggr = lax.dot_general(mean_vec, wc_ref[...],
                           (((1,), (1,)), ((), ()))) + bc_ref[...]
    rv_ref[...] = (srhs * w + b) - aggr


def _outer(s, vec):
    return lax.dot_general(s, vec, (((0,), (0,)), ((), ())),
                           preferred_element_type=jnp.float32)


def _emb_body(n, rhss_ref, w_ref, b_ref, emb_ref):
    emb_ref[...] = _outer(rhss_ref[0:1, :n], w_ref[...]) + b_ref[...]


def _final_body(n, xv0_ref, invv_ref, t1_ref, t2_ref, w_ref, b_ref, rv_ref,
                xvar_ref, xconst_ref):
    w = w_ref[...]
    b = b_ref[...]
    rv = rv_ref[...]
    xvar_ref[...] = (_outer(invv_ref[0:1, :n], rv * w)
                     + _outer(xv0_ref[0:1, :n], w) + (rv * b + b))
    xconst_ref[...] = (_outer(t1_ref[0:1, :n], w)
                      + _outer(t2_ref[0:1, :n], b))


def kernel(hyperedge_index, coef, rhs, W_rhs, b_rhs, W_c, b_c):
    f32 = jnp.float32
    nnz = coef.shape[0]
    n = rhs.shape[0]
    D = W_rhs.shape[0]
    e_per = nnz // NW
    assert nnz % NW == 0 and n <= NPAD

    he = hyperedge_index.astype(jnp.int32).reshape(2 * nnz)
    coef = coef.astype(f32)

    pass_a, pass_b, pass_c = _make_sc_kernels(e_per, nnz)

    sabs_p, csum_p = pass_a(coef, he)
    rhs_pad = jnp.pad(rhs[:, 0].astype(f32), (0, NPAD - n)).reshape(1, NPAD)
    inv_sc, t2, rhs_s = pl.pallas_call(
        _g1_body,
        out_shape=[jax.ShapeDtypeStruct((1, NPAD), f32)] * 3,
    )(sabs_p, csum_p, rhs_pad)

    w = W_rhs[:, 0].astype(f32).reshape(1, D)
    b = b_rhs.astype(f32).reshape(1, D)

    emb = pl.pallas_call(
        functools.partial(_emb_body, n),
        out_shape=jax.ShapeDtypeStruct((n, D), f32),
    )(rhs_s, w, b)

    sv_p, u_p = pass_b(coef, he, inv_sc.reshape(NPAD), rhs_s.reshape(NPAD))
    inv_sv, xv0 = pl.pallas_call(
        _g2_body,
        out_shape=[jax.ShapeDtypeStruct((1, NPAD), f32)] * 2,
    )(sv_p, u_p)

    (v_p,) = pass_c(coef, he, xv0.reshape(NPAD))
    t1, rhs_vec = pl.pallas_call(
        functools.partial(_g3_body, float(n)),
        out_shape=[jax.ShapeDtypeStruct((1, NPAD), f32),
                   jax.ShapeDtypeStruct((1, D), f32)],
    )(v_p, inv_sc, t2, rhs_s, W_c.astype(f32), b_c.astype(f32).reshape(1, D),
      w, b)

    xvar, xconst = pl.pallas_call(
        functools.partial(_final_body, n),
        out_shape=[jax.ShapeDtypeStruct((n, D), f32)] * 2,
    )(xv0, inv_sv, t1, t2, w, b, rhs_vec)

    return (xvar, xconst, emb)
